# Initial kernel scaffold; baseline (speedup 1.0000x reference)
import jax
import jax.numpy as jnp
from jax.experimental import pallas as pl


def kernel(x, a, gnn_W1, gnn_b1, gnn_g1, gnn_be1, gnn_W2, gnn_b2, gnn_g2, gnn_be2, gnn_eps, gru_Wih, gru_Whh, gru_bih, gru_bhh):
    raise NotImplementedError("write your pallas kernel here")


# speedup vs baseline: 1.8567x; 1.8567x over previous
import jax
import jax.numpy as jnp
from jax.experimental import pallas as pl
from jax.experimental.pallas import tpu as pltpu

B, ROI, T, H, F = 4, 90, 64, 3, 128
HID = 128
R = ROI * H
G = 3 * HID
BT = B * T
N_ROWS = BT * ROI
ROW_TILE = 512
N_TILES = N_ROWS // ROW_TILE
NRC = 5
RC = R // NRC


def _k1_body(a_ref, x_ref, w1_ref, b1_ref, eps_ref, z1_ref, st_ref, acc_ref):
    b = pl.program_id(1)
    t = pl.program_id(2)

    @pl.when(jnp.logical_and(b == 0, t == 0))
    def _():
        acc_ref[...] = jnp.zeros_like(acc_ref)

    adj = (a_ref[0, 0, 0] != 0.0).astype(jnp.float32)
    hx = x_ref[0, 0, 0]
    agg = jnp.dot(adj, hx, preferred_element_type=jnp.float32) + eps_ref[0, 0, 0] * hx
    z = jnp.dot(agg, w1_ref[0], preferred_element_type=jnp.float32) + b1_ref[0]
    z1_ref[0, 0, 0] = z
    acc_ref[0:1, :] += jnp.sum(z, axis=0, keepdims=True)
    acc_ref[1:2, :] += jnp.sum(z * z, axis=0, keepdims=True)

    @pl.when(jnp.logical_and(b == B - 1, t == T - 1))
    def _():
        st_ref[0] = acc_ref[...]


def _k2_body(z1_ref, sc1_ref, sh1_ref, w2_ref, b2_ref, z2_ref, st_ref, acc_ref):
    i = pl.program_id(1)

    @pl.when(i == 0)
    def _():
        acc_ref[...] = jnp.zeros_like(acc_ref)

    u = z1_ref[0] * sc1_ref[0] + sh1_ref[0]
    u = jnp.where(u > 0.0, u, jnp.exp(u) - 1.0)
    z2 = jnp.dot(u, w2_ref[0], preferred_element_type=jnp.float32) + b2_ref[0]
    z2_ref[0] = z2
    acc_ref[0:1, :] += jnp.sum(z2, axis=0, keepdims=True)
    acc_ref[1:2, :] += jnp.sum(z2 * z2, axis=0, keepdims=True)

    @pl.when(i == N_TILES - 1)
    def _():
        st_ref[0] = acc_ref[...]


def _k3_body(z2_ref, sc2_ref, sh2_ref, wih_ref, bih_ref, gi_ref):
    z = z2_ref[0, 0]
    u = jnp.maximum(z * sc2_ref[0] + sh2_ref[0], 0.0)
    gi = jnp.dot(u, wih_ref[0], preferred_element_type=jnp.float32) + bih_ref[0]
    gi_ref[:, 0] = gi.reshape(T, B, G)


def _k4_body(gi_ref, whh_ref, bhh_ref, y_ref, h_ref):
    t = pl.program_id(1)

    @pl.when(t == 0)
    def _():
        h_ref[...] = jnp.zeros_like(h_ref)

    hp = h_ref[...]
    gh = jax.lax.dot_general(
        hp, whh_ref[...],
        dimension_numbers=(((2,), (1,)), ((0,), (0,))),
        preferred_element_type=jnp.float32,
    ) + bhh_ref[...]
    gi = gi_ref[0]
    rg = jax.nn.sigmoid(gi[:, :, :HID] + gh[:, :, :HID])
    zg = jax.nn.sigmoid(gi[:, :, HID:2 * HID] + gh[:, :, HID:2 * HID])
    ng = jnp.tanh(gi[:, :, 2 * HID:] + rg * gh[:, :, 2 * HID:])
    hn = (1.0 - zg) * ng + zg * hp
    h_ref[...] = hn
    y_ref[0] = hn


def _bn_affine(st, g, be):
    n = jnp.float32(N_ROWS)
    mean = st[:, 0, :] / n
    var = st[:, 1, :] / n - mean * mean
    rstd = jax.lax.rsqrt(var + 1e-5)
    scale = g * rstd
    shift = be - mean * scale
    return scale.reshape(H, 1, F), shift.reshape(H, 1, F)


def kernel(x, a, gnn_W1, gnn_b1, gnn_g1, gnn_be1, gnn_W2, gnn_b2, gnn_g2,
           gnn_be2, gnn_eps, gru_Wih, gru_Whh, gru_bih, gru_bhh):
    a_t = jnp.transpose(a, (3, 0, 2, 1, 4))
    x_t = jnp.transpose(x, (3, 0, 2, 1, 4))

    z1, st1 = pl.pallas_call(
        _k1_body,
        grid=(H, B, T),
        in_specs=[
            pl.BlockSpec((1, 1, 1, ROI, ROI), lambda h, b, t: (h, b, t, 0, 0)),
            pl.BlockSpec((1, 1, 1, ROI, F), lambda h, b, t: (h, b, t, 0, 0)),
            pl.BlockSpec((1, F, F), lambda h, b, t: (h, 0, 0)),
            pl.BlockSpec((1, 1, F), lambda h, b, t: (h, 0, 0)),
            pl.BlockSpec((1, 1, 1), lambda h, b, t: (h, 0, 0)),
        ],
        out_specs=[
            pl.BlockSpec((1, 1, 1, ROI, F), lambda h, b, t: (h, b, t, 0, 0)),
            pl.BlockSpec((1, 8, F), lambda h, b, t: (h, 0, 0)),
        ],
        out_shape=[
            jax.ShapeDtypeStruct((H, B, T, ROI, F), jnp.float32),
            jax.ShapeDtypeStruct((H, 8, F), jnp.float32),
        ],
        scratch_shapes=[pltpu.VMEM((8, F), jnp.float32)],
    )(a_t, x_t, gnn_W1, gnn_b1.reshape(H, 1, F), gnn_eps.reshape(H, 1, 1))

    scale1, shift1 = _bn_affine(st1, gnn_g1, gnn_be1)
    z1f = jnp.transpose(z1, (0, 3, 2, 1, 4)).reshape(H, N_ROWS, F)

    z2f, st2 = pl.pallas_call(
        _k2_body,
        grid=(H, N_TILES),
        in_specs=[
            pl.BlockSpec((1, ROW_TILE, F), lambda h, i: (h, i, 0)),
            pl.BlockSpec((1, 1, F), lambda h, i: (h, 0, 0)),
            pl.BlockSpec((1, 1, F), lambda h, i: (h, 0, 0)),
            pl.BlockSpec((1, F, F), lambda h, i: (h, 0, 0)),
            pl.BlockSpec((1, 1, F), lambda h, i: (h, 0, 0)),
        ],
        out_specs=[
            pl.BlockSpec((1, ROW_TILE, F), lambda h, i: (h, i, 0)),
            pl.BlockSpec((1, 8, F), lambda h, i: (h, 0, 0)),
        ],
        out_shape=[
            jax.ShapeDtypeStruct((H, N_ROWS, F), jnp.float32),
            jax.ShapeDtypeStruct((H, 8, F), jnp.float32),
        ],
        scratch_shapes=[pltpu.VMEM((8, F), jnp.float32)],
    )(z1f, scale1, shift1, gnn_W2, gnn_b2.reshape(H, 1, F))

    scale2, shift2 = _bn_affine(st2, gnn_g2, gnn_be2)
    z2r = z2f.reshape(H, ROI, T * B, F)
    wih_t = jnp.transpose(gru_Wih, (0, 2, 1))

    gi = pl.pallas_call(
        _k3_body,
        grid=(H, ROI),
        in_specs=[
            pl.BlockSpec((1, 1, T * B, F), lambda h, r: (h, r, 0, 0)),
            pl.BlockSpec((1, 1, F), lambda h, r: (h, 0, 0)),
            pl.BlockSpec((1, 1, F), lambda h, r: (h, 0, 0)),
            pl.BlockSpec((1, F, G), lambda h, r: (r * H + h, 0, 0)),
            pl.BlockSpec((1, 1, G), lambda h, r: (r * H + h, 0, 0)),
        ],
        out_specs=pl.BlockSpec((T, 1, B, G), lambda h, r: (0, r * H + h, 0, 0)),
        out_shape=jax.ShapeDtypeStruct((T, R, B, G), jnp.float32),
    )(z2r, scale2, shift2, wih_t, gru_bih.reshape(R, 1, G))

    whh_t = jnp.transpose(gru_Whh, (0, 2, 1))

    y = pl.pallas_call(
        _k4_body,
        grid=(NRC, T),
        in_specs=[
            pl.BlockSpec((1, RC, B, G), lambda c, t: (t, c, 0, 0)),
            pl.BlockSpec((RC, HID, G), lambda c, t: (c, 0, 0)),
            pl.BlockSpec((RC, 1, G), lambda c, t: (c, 0, 0)),
        ],
        out_specs=pl.BlockSpec((1, RC, B, HID), lambda c, t: (t, c, 0, 0)),
        out_shape=jax.ShapeDtypeStruct((T, R, B, HID), jnp.float32),
        scratch_shapes=[pltpu.VMEM((RC, B, HID), jnp.float32)],
    )(gi, whh_t, gru_bhh.reshape(R, 1, G))

    yb = jnp.transpose(y, (2, 1, 0, 3)).reshape(B, ROI, H, T, HID)
    return jnp.transpose(yb, (0, 1, 3, 2, 4))
